# baseline (speedup 1.0000x reference)
import functools

import jax
import jax.numpy as jnp
from jax import lax
from jax.experimental import pallas as pl
from jax.experimental.pallas import tpu as pltpu
from jax.experimental.pallas import tpu_sc as plsc

_NC = 2
_NS = 16
_NW = _NC * _NS
_L = 16
_CH = 128


def _dist2_sc(pairs_flat, table):
    n_pairs = pairs_flat.shape[0] // 2
    d = table.shape[1]
    per_w = n_pairs // _NW
    rows_per_w = 2 * per_w
    n_ch = rows_per_w // _CH
    n_grp = per_w // _L

    mesh = plsc.VectorSubcoreMesh(core_axis_name="c", subcore_axis_name="s")

    @functools.partial(
        pl.kernel,
        out_type=jax.ShapeDtypeStruct((n_pairs,), jnp.float32),
        mesh=mesh,
        compiler_params=pltpu.CompilerParams(
            needs_layout_passes=False, use_tc_tiling_on_sc=False),
        scratch_types=[
            pltpu.VMEM((rows_per_w,), jnp.int32),
            pltpu.VMEM((rows_per_w, d), jnp.float32),
            pltpu.VMEM((per_w,), jnp.float32),
            pltpu.SemaphoreType.DMA,
        ],
    )
    def sc_kernel(pairs_hbm, table_hbm, out_hbm, idx_v, rows_v, d2_v, sem):
        wid = lax.axis_index("s") * _NC + lax.axis_index("c")
        pltpu.sync_copy(pairs_hbm.at[pl.ds(wid * rows_per_w, rows_per_w)],
                        idx_v)
        copies = [
            pltpu.async_copy(
                table_hbm.at[idx_v.at[pl.ds(c * _CH, _CH)]],
                rows_v.at[pl.ds(c * _CH, _CH), :],
                sem,
            )
            for c in range(n_ch)
        ]
        for cp in copies:
            cp.wait()

        lanes = lax.iota(jnp.int32, _L)

        def group_body(g, carry):
            row_u = g * (2 * _L) + 2 * lanes
            row_v = row_u + 1
            acc = jnp.zeros((_L,), jnp.float32)
            for j in range(d):
                col = jnp.full((_L,), j, jnp.int32)
                uu = plsc.load_gather(rows_v, [row_u, col])
                vv = plsc.load_gather(rows_v, [row_v, col])
                dd = uu - vv
                acc = acc + dd * dd
            d2_v[pl.ds(g * _L, _L)] = acc
            return carry

        lax.fori_loop(0, n_grp, group_body, 0)
        pltpu.sync_copy(d2_v, out_hbm.at[pl.ds(wid * per_w, per_w)])

    return sc_kernel(pairs_flat, table)


def _loss_tc(d2_mat, lbl_mat, bg):
    def body(bg_ref, d2_ref, lbl_ref, out_ref):
        beta = bg_ref[0]
        gamma = bg_ref[1]
        dist = jnp.sqrt(d2_ref[:])
        z = beta * dist - gamma
        s = jnp.where(lbl_ref[:] == 1, jnp.float32(1.0), jnp.float32(-1.0))
        out_ref[:] = jnp.logaddexp(jnp.float32(0.0), s * z)

    return pl.pallas_call(
        body,
        out_shape=jax.ShapeDtypeStruct(d2_mat.shape, jnp.float32),
        in_specs=[
            pl.BlockSpec(memory_space=pltpu.SMEM),
            pl.BlockSpec(memory_space=pltpu.VMEM),
            pl.BlockSpec(memory_space=pltpu.VMEM),
        ],
        out_specs=pl.BlockSpec(memory_space=pltpu.VMEM),
    )(bg, d2_mat, lbl_mat)


def kernel(pairs, labels, table, beta, gamma):
    n_pairs = pairs.shape[0]
    d2 = _dist2_sc(pairs.reshape(-1), table)
    rows = n_pairs // 128
    bg = jnp.stack([jnp.asarray(beta, jnp.float32),
                    jnp.asarray(gamma, jnp.float32)])
    loss = _loss_tc(d2.reshape(rows, 128), labels.reshape(rows, 128), bg)
    return loss.reshape(n_pairs)
